# Initial kernel scaffold; baseline (speedup 1.0000x reference)
import jax
import jax.numpy as jnp
from jax.experimental import pallas as pl


def kernel(x, edge_index, ex, batch, atom_emb, bond_emb, conv_eps, conv_w1, conv_b1, conv_bng, conv_bnb, conv_w2, conv_b2, bn_g, bn_b, vn_emb, vn_w1, vn_b1, vn_bn1g, vn_bn1b, vn_w2, vn_b2, vn_bn2g, vn_bn2b, out_w, out_b):
    raise NotImplementedError("write your pallas kernel here")


# speedup vs baseline: 11.8941x; 11.8941x over previous
import functools

import jax
import jax.numpy as jnp
from jax import lax
from jax.experimental import pallas as pl
from jax.experimental.pallas import tpu as pltpu
from jax.experimental.pallas import tpu_sc as plsc

N = 10000
NP = 10240
E = 320000
H = 128
H2 = 256
NG = 128
L = 3

BN = 1024
GRID = NP // BN

NCORE = 2
NSUB = 16
TILES = NCORE * NSUB
PTE = E // TILES
WSZ = 128
WPT = (PTE + WSZ - 1) // WSZ
PAD = WPT * WSZ - PTE
ROWS_PER_TILE = NP // NSUB


def _sc_aggr_body(haug, epk, out, ev, idxv, dstv, rows, shared, sem):
    c = lax.axis_index("c")
    s = lax.axis_index("s")
    wid = c * NSUB + s

    def zrow(r, _):
        for k in range(H // 16):
            rows[r, pl.ds(16 * k, 16)] = jnp.zeros((16,), jnp.float32)
        return 0
    lax.fori_loop(0, WSZ, zrow, 0)

    def zcp(j, _):
        pltpu.sync_copy(rows, shared.at[pl.ds(s * ROWS_PER_TILE + j * WSZ, WSZ)])
        return 0
    lax.fori_loop(0, ROWS_PER_TILE // WSZ, zcp, 0)
    plsc.subcore_barrier()

    def win(w, _):
        pltpu.sync_copy(epk.at[wid * WPT + w], ev)
        for k in range(WSZ // 16):
            sl = pl.ds(16 * k, 16)
            idxv[sl] = ev[0, sl] * 8 + ev[1, sl]
            dstv[sl] = ev[2, sl]
        pltpu.async_copy(haug.at[idxv], rows, sem).wait()
        pltpu.sync_copy(rows, shared.at[dstv], add=True)
        return 0
    lax.fori_loop(0, WPT, win, 0)

    plsc.subcore_barrier()
    pltpu.sync_copy(shared.at[pl.ds(s * ROWS_PER_TILE, ROWS_PER_TILE)],
                    out.at[c, pl.ds(s * ROWS_PER_TILE, ROWS_PER_TILE)])


@functools.cache
def _make_sc_aggr():
    return pl.kernel(
        _sc_aggr_body,
        out_type=jax.ShapeDtypeStruct((NCORE, NP, H), jnp.float32),
        mesh=plsc.VectorSubcoreMesh(core_axis_name="c", subcore_axis_name="s",
                                    num_cores=NCORE, num_subcores=NSUB),
        scratch_types=[
            pltpu.VMEM((3, WSZ), jnp.int32),
            pltpu.VMEM((WSZ,), jnp.int32),
            pltpu.VMEM((WSZ,), jnp.int32),
            pltpu.VMEM((WSZ, H), jnp.float32),
            pltpu.VMEM_SHARED((NP, H), jnp.float32),
            pltpu.SemaphoreType.DMA,
        ],
    )


def _sc_aggr(haug, epk):
    return _make_sc_aggr()(haug, epk)


def _enc_body(xf_ref, d_ref, b_ref, o_ref):
    o_ref[...] = (jnp.dot(xf_ref[...], d_ref[...],
                          preferred_element_type=jnp.float32)
                  + b_ref[...])


def _enc(xfp, datom, bias):
    return pl.pallas_call(
        _enc_body,
        out_shape=jax.ShapeDtypeStruct((NP, H), jnp.float32),
    )(xfp, datom, bias)


def _build_body(h_ref, t_ref, o_ref):
    o_ref[...] = jnp.maximum(h_ref[...][:, None, :] + t_ref[...][None, :, :],
                             0.0)


def _build(h, t):
    return pl.pallas_call(
        _build_body,
        grid=(GRID,),
        in_specs=[
            pl.BlockSpec((BN, H), lambda i: (i, 0)),
            pl.BlockSpec((8, H), lambda i: (0, 0)),
        ],
        out_specs=pl.BlockSpec((BN, 8, H), lambda i: (i, 0, 0)),
        out_shape=jax.ShapeDtypeStruct((NP, 8, H), jnp.float32),
    )(h, t)


def _build_vn_body(h_ref, b2d_ref, vx_ref, t_ref, hc_ref, o_ref):
    oh = (b2d_ref[...] == lax.broadcasted_iota(jnp.int32, (1, NG), 1)
          ).astype(jnp.float32)
    hc = h_ref[...] + jnp.dot(oh, vx_ref[...],
                              preferred_element_type=jnp.float32)
    hc_ref[...] = hc
    o_ref[...] = jnp.maximum(hc[:, None, :] + t_ref[...][None, :, :], 0.0)


def _build_vn(h, b2d, vx, t):
    return pl.pallas_call(
        _build_vn_body,
        grid=(GRID,),
        in_specs=[
            pl.BlockSpec((BN, H), lambda i: (i, 0)),
            pl.BlockSpec((BN, 1), lambda i: (i, 0)),
            pl.BlockSpec((NG, H), lambda i: (0, 0)),
            pl.BlockSpec((8, H), lambda i: (0, 0)),
        ],
        out_specs=[
            pl.BlockSpec((BN, H), lambda i: (i, 0)),
            pl.BlockSpec((BN, 8, H), lambda i: (i, 0, 0)),
        ],
        out_shape=[
            jax.ShapeDtypeStruct((NP, H), jnp.float32),
            jax.ShapeDtypeStruct((NP, 8, H), jnp.float32),
        ],
    )(h, b2d, vx, t)


def _conv_body(h_ref, p_ref, w1_ref, b1_ref, w2_ref, b2_ref, e_ref, o_ref,
               *, relu_out):
    h2 = h_ref[...] * e_ref[0, 0] + p_ref[0] + p_ref[1]
    m = jnp.maximum(jnp.dot(h2, w1_ref[...],
                            preferred_element_type=jnp.float32)
                    + b1_ref[...], 0.0)
    z = (jnp.dot(m, w2_ref[...], preferred_element_type=jnp.float32)
         + b2_ref[...])
    o_ref[...] = jnp.maximum(z, 0.0) if relu_out else z


def _conv(h, p, w1f, b1f, w2f, b2f, eps, relu_out):
    return pl.pallas_call(
        functools.partial(_conv_body, relu_out=relu_out),
        grid=(GRID,),
        in_specs=[
            pl.BlockSpec((BN, H), lambda i: (i, 0)),
            pl.BlockSpec((NCORE, BN, H), lambda i: (0, i, 0)),
            pl.BlockSpec((H, H2), lambda i: (0, 0)),
            pl.BlockSpec((1, H2), lambda i: (0, 0)),
            pl.BlockSpec((H2, H), lambda i: (0, 0)),
            pl.BlockSpec((1, H), lambda i: (0, 0)),
            pl.BlockSpec((1, 1), lambda i: (0, 0)),
        ],
        out_specs=pl.BlockSpec((BN, H), lambda i: (i, 0)),
        out_shape=jax.ShapeDtypeStruct((NP, H), jnp.float32),
    )(h, p, w1f, b1f, w2f, b2f, eps)


def _vnpool_body(h_ref, b2d_ref, vn0_ref, w1_ref, b1_ref, w2_ref, b2_ref,
                 o_ref, acc):
    i = pl.program_id(0)
    oh = (b2d_ref[...] == lax.broadcasted_iota(jnp.int32, (1, NG), 1)
          ).astype(jnp.float32)
    part = lax.dot_general(oh, h_ref[...], (((0,), (0,)), ((), ())),
                           preferred_element_type=jnp.float32)

    @pl.when(i == 0)
    def _():
        acc[...] = part

    @pl.when(i > 0)
    def _():
        acc[...] = acc[...] + part

    @pl.when(i == GRID - 1)
    def _():
        row = acc[...] + vn0_ref[...]
        t = jnp.maximum(jnp.dot(row, w1_ref[...],
                                preferred_element_type=jnp.float32)
                        + b1_ref[...], 0.0)
        o_ref[...] = jnp.maximum(jnp.dot(t, w2_ref[...],
                                         preferred_element_type=jnp.float32)
                                 + b2_ref[...], 0.0)


def _vnpool(h, b2d, vn0, vw1f, vb1f, vw2f, vb2f):
    return pl.pallas_call(
        _vnpool_body,
        grid=(GRID,),
        in_specs=[
            pl.BlockSpec((BN, H), lambda i: (i, 0)),
            pl.BlockSpec((BN, 1), lambda i: (i, 0)),
            pl.BlockSpec((1, H), lambda i: (0, 0)),
            pl.BlockSpec((H, H2), lambda i: (0, 0)),
            pl.BlockSpec((1, H2), lambda i: (0, 0)),
            pl.BlockSpec((H2, H), lambda i: (0, 0)),
            pl.BlockSpec((1, H), lambda i: (0, 0)),
        ],
        out_specs=pl.BlockSpec((NG, H), lambda i: (0, 0)),
        out_shape=jax.ShapeDtypeStruct((NG, H), jnp.float32),
        scratch_shapes=[pltpu.VMEM((NG, H), jnp.float32)],
    )(h, b2d, vn0, vw1f, vb1f, vw2f, vb2f)


def _final_body(h_ref, b2d_ref, w_ref, b_ref, o_ref, acc, cnt):
    i = pl.program_id(0)
    oh = (b2d_ref[...] == lax.broadcasted_iota(jnp.int32, (1, NG), 1)
          ).astype(jnp.float32)
    part = lax.dot_general(oh, h_ref[...], (((0,), (0,)), ((), ())),
                           preferred_element_type=jnp.float32)
    ones = jnp.ones((BN, 1), jnp.float32)
    cpart = lax.dot_general(oh, ones, (((0,), (0,)), ((), ())),
                            preferred_element_type=jnp.float32)

    @pl.when(i == 0)
    def _():
        acc[...] = part
        cnt[...] = cpart

    @pl.when(i > 0)
    def _():
        acc[...] = acc[...] + part
        cnt[...] = cnt[...] + cpart

    @pl.when(i == GRID - 1)
    def _():
        hg = acc[...] / jnp.maximum(cnt[...], 1.0)
        o_ref[...] = (jnp.dot(hg, w_ref[...],
                              preferred_element_type=jnp.float32)
                      + b_ref[...])


def _final(h, b2d, out_w, out_b):
    return pl.pallas_call(
        _final_body,
        grid=(GRID,),
        in_specs=[
            pl.BlockSpec((BN, H), lambda i: (i, 0)),
            pl.BlockSpec((BN, 1), lambda i: (i, 0)),
            pl.BlockSpec((H, H), lambda i: (0, 0)),
            pl.BlockSpec((1, H), lambda i: (0, 0)),
        ],
        out_specs=pl.BlockSpec((NG, H), lambda i: (0, 0)),
        out_shape=jax.ShapeDtypeStruct((NG, H), jnp.float32),
        scratch_shapes=[pltpu.VMEM((NG, H), jnp.float32),
                        pltpu.VMEM((NG, 1), jnp.float32)],
    )(h, b2d, out_w, out_b)


def kernel(x, edge_index, ex, batch, atom_emb, bond_emb, conv_eps, conv_w1,
           conv_b1, conv_bng, conv_bnb, conv_w2, conv_b2, bn_g, bn_b, vn_emb,
           vn_w1, vn_b1, vn_bn1g, vn_bn1b, vn_w2, vn_b2, vn_bn2g, vn_bn2b,
           out_w, out_b):
    f32 = jnp.float32
    rs = 1.0 / jnp.sqrt(1.0 + 1e-5)

    datom = atom_emb[:, 1, :] - atom_emb[:, 0, :]
    encb = (atom_emb[:, 0, :].sum(0) + vn_emb[0])[None, :]
    combos = jnp.array([[c & 1, (c >> 1) & 1, (c >> 2) & 1]
                        for c in range(8)], f32)
    dbond = bond_emb[:, :, 1, :] - bond_emb[:, :, 0, :]
    cbond = bond_emb[:, :, 0, :].sum(1)
    tables = cbond[:, None, :] + jnp.einsum("cj,ljh->lch", combos, dbond)

    s1 = conv_bng * rs
    w1f = conv_w1 * s1[:, None, :]
    b1f = conv_b1 * s1 + conv_bnb
    s2 = jnp.concatenate([bn_g * rs, jnp.ones((1, H), f32)], 0)
    badd = jnp.concatenate([bn_b, jnp.zeros((1, H), f32)], 0)
    w2f = conv_w2 * s2[:, None, :]
    b2f = conv_b2 * s2 + badd
    sv1 = vn_bn1g[0] * rs
    vw1f = vn_w1[0] * sv1[None, :]
    vb1f = (vn_b1[0] * sv1 + vn_bn1b[0])[None, :]
    sv2 = vn_bn2g[0] * rs
    vw2f = vn_w2[0] * sv2[None, :]
    vb2f = (vn_b2[0] * sv2 + vn_bn2b[0])[None, :]
    eps1 = (1.0 + conv_eps).astype(f32)

    xfp = jnp.zeros((NP, 9), f32).at[:N].set(x.astype(f32))
    batchpad = jnp.concatenate(
        [batch.astype(jnp.int32), jnp.full((NP - N,), -1, jnp.int32)])
    b2d = batchpad.reshape(NP, 1)

    src = edge_index[0].astype(jnp.int32)
    dst = edge_index[1].astype(jnp.int32)
    code = (ex[:, 0] + 2 * ex[:, 1] + 4 * ex[:, 2]).astype(jnp.int32)
    padsrc = (jnp.arange(PAD, dtype=jnp.int32) * 89) % N
    paddst = N + (jnp.arange(PAD, dtype=jnp.int32) % (NP - N))
    padcode = jnp.zeros((PAD,), jnp.int32)

    def tile_pack(a, padvals):
        a2 = a.reshape(TILES, PTE)
        p = jnp.broadcast_to(padvals, (TILES, PAD))
        return jnp.concatenate([a2, p], axis=1).reshape(TILES * WPT, WSZ)

    epk = jnp.stack([tile_pack(src, padsrc), tile_pack(code, padcode),
                     tile_pack(dst, paddst)], axis=1)

    h_a = _enc(xfp, datom, encb)
    p = _sc_aggr(_build(h_a, tables[0]).reshape(NP * 8, H), epk)
    h_b = _conv(h_a, p, w1f[0], b1f[0][None], w2f[0], b2f[0][None],
                eps1[0].reshape(1, 1), relu_out=True)
    vx = _vnpool(h_b, b2d, vn_emb[0][None], vw1f, vb1f, vw2f, vb2f)
    h_c, aug1 = _build_vn(h_b, b2d, vx, tables[1])
    p = _sc_aggr(aug1.reshape(NP * 8, H), epk)
    h_d = _conv(h_c, p, w1f[1], b1f[1][None], w2f[1], b2f[1][None],
                eps1[1].reshape(1, 1), relu_out=True)
    p = _sc_aggr(_build(h_d, tables[2]).reshape(NP * 8, H), epk)
    h_e = _conv(h_d, p, w1f[2], b1f[2][None], w2f[2], b2f[2][None],
                eps1[2].reshape(1, 1), relu_out=False)
    return _final(h_e, b2d, out_w, out_b[None, :])


# speedup vs baseline: 18.8123x; 1.5817x over previous
import functools

import jax
import jax.numpy as jnp
from jax import lax
from jax.experimental import pallas as pl
from jax.experimental.pallas import tpu as pltpu
from jax.experimental.pallas import tpu_sc as plsc

N = 10000
NP = 10240
E = 320000
H = 128
H2 = 256
NG = 128
L = 3

BN = 1024
GRID = NP // BN

NCORE = 2
NSUB = 16
TILES = NCORE * NSUB
PTE = E // TILES
WSZ = 128
WPT = 80
PAD = WPT * WSZ - PTE
ROWS_PER_TILE = NP // NSUB
NRING = 2
EVW = 4
WBLK = 8
NBLK = WPT // WBLK


def _sc_aggr_body(haug, epk, out, ev, idxv, dstv, rows, shared,
                  gs0, gs1, ss0, ss1, es0, es1):
    c = lax.axis_index("c")
    s = lax.axis_index("s")
    wid = c * NSUB + s
    base = wid * WPT
    gsem = (gs0, gs1)
    ssem = (ss0, ss1)
    esem = (es0, es1)

    def zrow(r, _):
        for k in range(H // 16):
            rows[0, r, pl.ds(16 * k, 16)] = jnp.zeros((16,), jnp.float32)
        return 0
    lax.fori_loop(0, WSZ, zrow, 0)

    def zcp(j, _):
        pltpu.sync_copy(rows.at[0],
                        shared.at[pl.ds(s * ROWS_PER_TILE + j * WSZ, WSZ)])
        return 0
    lax.fori_loop(0, ROWS_PER_TILE // WSZ, zcp, 0)
    plsc.subcore_barrier()

    def ev_load(blk, slot, sem):
        return pltpu.async_copy(
            epk.at[pl.ds(base + blk * WBLK + slot * EVW, EVW)],
            ev.at[slot], sem)

    def gather(b):
        return pltpu.async_copy(haug.at[idxv.at[b]], rows.at[b], gsem[b])

    def scatter(b):
        return pltpu.async_copy(rows.at[b], shared.at[dstv.at[b]], ssem[b],
                                add=True)

    def wait_ev(slot):
        pltpu.make_async_copy(epk.at[pl.ds(base, EVW)], ev.at[slot],
                              esem[slot]).wait()

    def wait_g(b):
        pltpu.make_async_copy(haug.at[pl.ds(0, WSZ)], rows.at[b],
                              gsem[b]).wait()

    def wait_s(b):
        pltpu.make_async_copy(haug.at[pl.ds(0, WSZ)], rows.at[b],
                              ssem[b]).wait()

    ev_load(0, 0, esem[0])
    ev_load(0, 1, esem[1])

    def blk(gg, _):
        first = gg == 0
        not_last = gg < NBLK - 1
        for j in range(WBLK):
            b = j % NRING
            slot = j // EVW
            if j == 0:
                wait_ev(0)
            if j == EVW:
                wait_ev(1)
            if j < NRING:
                @pl.when(~first)
                def _():
                    wait_s(b)
            else:
                wait_s(b)
            for k in range(WSZ // 16):
                sl = pl.ds(16 * k, 16)
                idxv[b, sl] = ev[slot, j - slot * EVW, 0, sl] * 8 \
                    + ev[slot, j - slot * EVW, 1, sl]
                dstv[b, sl] = ev[slot, j - slot * EVW, 2, sl]
            if j == EVW - 1:
                @pl.when(not_last)
                def _():
                    ev_load(gg + 1, 0, esem[0])
            if j == WBLK - 1:
                @pl.when(not_last)
                def _():
                    ev_load(gg + 1, 1, esem[1])
            gather(b)
            pb = (b - 1) % NRING
            if j == 0:
                @pl.when(~first)
                def _():
                    wait_g(pb)
                    scatter(pb)
            else:
                wait_g(pb)
                scatter(pb)
        return 0
    lax.fori_loop(0, NBLK, blk, 0)

    wait_g((WPT - 1) % NRING)
    scatter((WPT - 1) % NRING)
    for b in range(NRING):
        wait_s(b)

    plsc.subcore_barrier()
    pltpu.sync_copy(shared.at[pl.ds(s * ROWS_PER_TILE, ROWS_PER_TILE)],
                    out.at[c, pl.ds(s * ROWS_PER_TILE, ROWS_PER_TILE)])


@functools.cache
def _make_sc_aggr():
    return pl.kernel(
        _sc_aggr_body,
        out_type=jax.ShapeDtypeStruct((NCORE, NP, H), jnp.float32),
        mesh=plsc.VectorSubcoreMesh(core_axis_name="c", subcore_axis_name="s",
                                    num_cores=NCORE, num_subcores=NSUB),
        scratch_types=(
            [
                pltpu.VMEM((2, EVW, 3, WSZ), jnp.int32),
                pltpu.VMEM((NRING, WSZ), jnp.int32),
                pltpu.VMEM((NRING, WSZ), jnp.int32),
                pltpu.VMEM((NRING, WSZ, H), jnp.float32),
                pltpu.VMEM_SHARED((NP, H), jnp.float32),
            ]
            + [pltpu.SemaphoreType.DMA] * 6
        ),
    )


def _sc_aggr(haug, epk):
    return _make_sc_aggr()(haug, epk)


def _enc_body(xf_ref, d_ref, b_ref, o_ref):
    o_ref[...] = (jnp.dot(xf_ref[...], d_ref[...],
                          preferred_element_type=jnp.float32)
                  + b_ref[...])


def _enc(xfp, datom, bias):
    return pl.pallas_call(
        _enc_body,
        out_shape=jax.ShapeDtypeStruct((NP, H), jnp.float32),
    )(xfp, datom, bias)


def _build_body(h_ref, t_ref, o_ref):
    o_ref[...] = jnp.maximum(h_ref[...][:, None, :] + t_ref[...][None, :, :],
                             0.0)


def _build(h, t):
    return pl.pallas_call(
        _build_body,
        grid=(GRID,),
        in_specs=[
            pl.BlockSpec((BN, H), lambda i: (i, 0)),
            pl.BlockSpec((8, H), lambda i: (0, 0)),
        ],
        out_specs=pl.BlockSpec((BN, 8, H), lambda i: (i, 0, 0)),
        out_shape=jax.ShapeDtypeStruct((NP, 8, H), jnp.float32),
    )(h, t)


def _build_vn_body(h_ref, b2d_ref, vx_ref, t_ref, hc_ref, o_ref):
    oh = (b2d_ref[...] == lax.broadcasted_iota(jnp.int32, (1, NG), 1)
          ).astype(jnp.float32)
    hc = h_ref[...] + jnp.dot(oh, vx_ref[...],
                              preferred_element_type=jnp.float32)
    hc_ref[...] = hc
    o_ref[...] = jnp.maximum(hc[:, None, :] + t_ref[...][None, :, :], 0.0)


def _build_vn(h, b2d, vx, t):
    return pl.pallas_call(
        _build_vn_body,
        grid=(GRID,),
        in_specs=[
            pl.BlockSpec((BN, H), lambda i: (i, 0)),
            pl.BlockSpec((BN, 1), lambda i: (i, 0)),
            pl.BlockSpec((NG, H), lambda i: (0, 0)),
            pl.BlockSpec((8, H), lambda i: (0, 0)),
        ],
        out_specs=[
            pl.BlockSpec((BN, H), lambda i: (i, 0)),
            pl.BlockSpec((BN, 8, H), lambda i: (i, 0, 0)),
        ],
        out_shape=[
            jax.ShapeDtypeStruct((NP, H), jnp.float32),
            jax.ShapeDtypeStruct((NP, 8, H), jnp.float32),
        ],
    )(h, b2d, vx, t)


def _conv_body(h_ref, p_ref, w1_ref, b1_ref, w2_ref, b2_ref, e_ref, o_ref,
               *, relu_out):
    h2 = h_ref[...] * e_ref[0, 0] + p_ref[0] + p_ref[1]
    m = jnp.maximum(jnp.dot(h2, w1_ref[...],
                            preferred_element_type=jnp.float32)
                    + b1_ref[...], 0.0)
    z = (jnp.dot(m, w2_ref[...], preferred_element_type=jnp.float32)
         + b2_ref[...])
    o_ref[...] = jnp.maximum(z, 0.0) if relu_out else z


def _conv(h, p, w1f, b1f, w2f, b2f, eps, relu_out):
    return pl.pallas_call(
        functools.partial(_conv_body, relu_out=relu_out),
        grid=(GRID,),
        in_specs=[
            pl.BlockSpec((BN, H), lambda i: (i, 0)),
            pl.BlockSpec((NCORE, BN, H), lambda i: (0, i, 0)),
            pl.BlockSpec((H, H2), lambda i: (0, 0)),
            pl.BlockSpec((1, H2), lambda i: (0, 0)),
            pl.BlockSpec((H2, H), lambda i: (0, 0)),
            pl.BlockSpec((1, H), lambda i: (0, 0)),
            pl.BlockSpec((1, 1), lambda i: (0, 0)),
        ],
        out_specs=pl.BlockSpec((BN, H), lambda i: (i, 0)),
        out_shape=jax.ShapeDtypeStruct((NP, H), jnp.float32),
    )(h, p, w1f, b1f, w2f, b2f, eps)


def _vnpool_body(h_ref, b2d_ref, vn0_ref, w1_ref, b1_ref, w2_ref, b2_ref,
                 o_ref, acc):
    i = pl.program_id(0)
    oh = (b2d_ref[...] == lax.broadcasted_iota(jnp.int32, (1, NG), 1)
          ).astype(jnp.float32)
    part = lax.dot_general(oh, h_ref[...], (((0,), (0,)), ((), ())),
                           preferred_element_type=jnp.float32)

    @pl.when(i == 0)
    def _():
        acc[...] = part

    @pl.when(i > 0)
    def _():
        acc[...] = acc[...] + part

    @pl.when(i == GRID - 1)
    def _():
        row = acc[...] + vn0_ref[...]
        t = jnp.maximum(jnp.dot(row, w1_ref[...],
                                preferred_element_type=jnp.float32)
                        + b1_ref[...], 0.0)
        o_ref[...] = jnp.maximum(jnp.dot(t, w2_ref[...],
                                         preferred_element_type=jnp.float32)
                                 + b2_ref[...], 0.0)


def _vnpool(h, b2d, vn0, vw1f, vb1f, vw2f, vb2f):
    return pl.pallas_call(
        _vnpool_body,
        grid=(GRID,),
        in_specs=[
            pl.BlockSpec((BN, H), lambda i: (i, 0)),
            pl.BlockSpec((BN, 1), lambda i: (i, 0)),
            pl.BlockSpec((1, H), lambda i: (0, 0)),
            pl.BlockSpec((H, H2), lambda i: (0, 0)),
            pl.BlockSpec((1, H2), lambda i: (0, 0)),
            pl.BlockSpec((H2, H), lambda i: (0, 0)),
            pl.BlockSpec((1, H), lambda i: (0, 0)),
        ],
        out_specs=pl.BlockSpec((NG, H), lambda i: (0, 0)),
        out_shape=jax.ShapeDtypeStruct((NG, H), jnp.float32),
        scratch_shapes=[pltpu.VMEM((NG, H), jnp.float32)],
    )(h, b2d, vn0, vw1f, vb1f, vw2f, vb2f)


def _final_body(h_ref, b2d_ref, w_ref, b_ref, o_ref, acc, cnt):
    i = pl.program_id(0)
    oh = (b2d_ref[...] == lax.broadcasted_iota(jnp.int32, (1, NG), 1)
          ).astype(jnp.float32)
    part = lax.dot_general(oh, h_ref[...], (((0,), (0,)), ((), ())),
                           preferred_element_type=jnp.float32)
    ones = jnp.ones((BN, 1), jnp.float32)
    cpart = lax.dot_general(oh, ones, (((0,), (0,)), ((), ())),
                            preferred_element_type=jnp.float32)

    @pl.when(i == 0)
    def _():
        acc[...] = part
        cnt[...] = cpart

    @pl.when(i > 0)
    def _():
        acc[...] = acc[...] + part
        cnt[...] = cnt[...] + cpart

    @pl.when(i == GRID - 1)
    def _():
        hg = acc[...] / jnp.maximum(cnt[...], 1.0)
        o_ref[...] = (jnp.dot(hg, w_ref[...],
                              preferred_element_type=jnp.float32)
                      + b_ref[...])


def _final(h, b2d, out_w, out_b):
    return pl.pallas_call(
        _final_body,
        grid=(GRID,),
        in_specs=[
            pl.BlockSpec((BN, H), lambda i: (i, 0)),
            pl.BlockSpec((BN, 1), lambda i: (i, 0)),
            pl.BlockSpec((H, H), lambda i: (0, 0)),
            pl.BlockSpec((1, H), lambda i: (0, 0)),
        ],
        out_specs=pl.BlockSpec((NG, H), lambda i: (0, 0)),
        out_shape=jax.ShapeDtypeStruct((NG, H), jnp.float32),
        scratch_shapes=[pltpu.VMEM((NG, H), jnp.float32),
                        pltpu.VMEM((NG, 1), jnp.float32)],
    )(h, b2d, out_w, out_b)


def kernel(x, edge_index, ex, batch, atom_emb, bond_emb, conv_eps, conv_w1,
           conv_b1, conv_bng, conv_bnb, conv_w2, conv_b2, bn_g, bn_b, vn_emb,
           vn_w1, vn_b1, vn_bn1g, vn_bn1b, vn_w2, vn_b2, vn_bn2g, vn_bn2b,
           out_w, out_b):
    f32 = jnp.float32
    rs = 1.0 / jnp.sqrt(1.0 + 1e-5)

    datom = atom_emb[:, 1, :] - atom_emb[:, 0, :]
    encb = (atom_emb[:, 0, :].sum(0) + vn_emb[0])[None, :]
    combos = jnp.array([[c & 1, (c >> 1) & 1, (c >> 2) & 1]
                        for c in range(8)], f32)
    dbond = bond_emb[:, :, 1, :] - bond_emb[:, :, 0, :]
    cbond = bond_emb[:, :, 0, :].sum(1)
    tables = cbond[:, None, :] + jnp.einsum("cj,ljh->lch", combos, dbond)

    s1 = conv_bng * rs
    w1f = conv_w1 * s1[:, None, :]
    b1f = conv_b1 * s1 + conv_bnb
    s2 = jnp.concatenate([bn_g * rs, jnp.ones((1, H), f32)], 0)
    badd = jnp.concatenate([bn_b, jnp.zeros((1, H), f32)], 0)
    w2f = conv_w2 * s2[:, None, :]
    b2f = conv_b2 * s2 + badd
    sv1 = vn_bn1g[0] * rs
    vw1f = vn_w1[0] * sv1[None, :]
    vb1f = (vn_b1[0] * sv1 + vn_bn1b[0])[None, :]
    sv2 = vn_bn2g[0] * rs
    vw2f = vn_w2[0] * sv2[None, :]
    vb2f = (vn_b2[0] * sv2 + vn_bn2b[0])[None, :]
    eps1 = (1.0 + conv_eps).astype(f32)

    xfp = jnp.zeros((NP, 9), f32).at[:N].set(x.astype(f32))
    batchpad = jnp.concatenate(
        [batch.astype(jnp.int32), jnp.full((NP - N,), -1, jnp.int32)])
    b2d = batchpad.reshape(NP, 1)

    src = edge_index[0].astype(jnp.int32)
    dst = edge_index[1].astype(jnp.int32)
    code = (ex[:, 0] + 2 * ex[:, 1] + 4 * ex[:, 2]).astype(jnp.int32)
    padsrc = (jnp.arange(PAD, dtype=jnp.int32) * 89) % N
    paddst = N + (jnp.arange(PAD, dtype=jnp.int32) % (NP - N))
    padcode = jnp.zeros((PAD,), jnp.int32)

    def tile_pack(a, padvals):
        a2 = a.reshape(TILES, PTE)
        p = jnp.broadcast_to(padvals, (TILES, PAD))
        return jnp.concatenate([a2, p], axis=1).reshape(TILES * WPT, WSZ)

    epk = jnp.stack([tile_pack(src, padsrc), tile_pack(code, padcode),
                     tile_pack(dst, paddst)], axis=1)

    h_a = _enc(xfp, datom, encb)
    p = _sc_aggr(_build(h_a, tables[0]).reshape(NP * 8, H), epk)
    h_b = _conv(h_a, p, w1f[0], b1f[0][None], w2f[0], b2f[0][None],
                eps1[0].reshape(1, 1), relu_out=True)
    vx = _vnpool(h_b, b2d, vn_emb[0][None], vw1f, vb1f, vw2f, vb2f)
    h_c, aug1 = _build_vn(h_b, b2d, vx, tables[1])
    p = _sc_aggr(aug1.reshape(NP * 8, H), epk)
    h_d = _conv(h_c, p, w1f[1], b1f[1][None], w2f[1], b2f[1][None],
                eps1[1].reshape(1, 1), relu_out=True)
    p = _sc_aggr(_build(h_d, tables[2]).reshape(NP * 8, H), epk)
    h_e = _conv(h_d, p, w1f[2], b1f[2][None], w2f[2], b2f[2][None],
                eps1[2].reshape(1, 1), relu_out=False)
    return _final(h_e, b2d, out_w, out_b[None, :])


# speedup vs baseline: 20.7819x; 1.1047x over previous
import functools

import jax
import jax.numpy as jnp
from jax import lax
from jax.experimental import pallas as pl
from jax.experimental.pallas import tpu as pltpu
from jax.experimental.pallas import tpu_sc as plsc

N = 10000
NP = 10240
E = 320000
H = 128
H2 = 256
NG = 128
L = 3

BN = 1024
GRID = NP // BN

NCORE = 2
NSUB = 16
TILES = NCORE * NSUB
PTE = E // TILES
WSZ = 128
WPT = 80
PAD = WPT * WSZ - PTE
ROWS_PER_TILE = NP // NSUB
NRING = 2
EVW = 4
WBLK = 8
NBLK = WPT // WBLK


def _sc_aggr_body(haug, epk, out, ev, idxv, dstv, rows, shared,
                  gs0, gs1, ss0, ss1, es0, es1):
    c = lax.axis_index("c")
    s = lax.axis_index("s")
    wid = c * NSUB + s
    base = wid * WPT
    gsem = (gs0, gs1)
    ssem = (ss0, ss1)
    esem = (es0, es1)

    def zrow(r, _):
        for k in range(H // 16):
            rows[0, r, pl.ds(16 * k, 16)] = jnp.zeros((16,), jnp.float32)
        return 0
    lax.fori_loop(0, WSZ, zrow, 0)

    for j in range(ROWS_PER_TILE // WSZ):
        pltpu.async_copy(rows.at[0],
                         shared.at[pl.ds(s * ROWS_PER_TILE + j * WSZ, WSZ)],
                         ss0)
    for j in range(ROWS_PER_TILE // WSZ):
        pltpu.make_async_copy(haug.at[pl.ds(0, WSZ)], rows.at[0], ss0).wait()
    plsc.subcore_barrier()

    def ev_load(blk, slot, sem):
        return pltpu.async_copy(
            epk.at[pl.ds(base + blk * WBLK + slot * EVW, EVW)],
            ev.at[slot], sem)

    def gather(b):
        return pltpu.async_copy(haug.at[idxv.at[b]], rows.at[b], gsem[b])

    def scatter(b):
        return pltpu.async_copy(rows.at[b], shared.at[dstv.at[b]], ssem[b],
                                add=True)

    def wait_ev(slot):
        pltpu.make_async_copy(epk.at[pl.ds(base, EVW)], ev.at[slot],
                              esem[slot]).wait()

    def wait_g(b):
        pltpu.make_async_copy(haug.at[pl.ds(0, WSZ)], rows.at[b],
                              gsem[b]).wait()

    def wait_s(b):
        pltpu.make_async_copy(haug.at[pl.ds(0, WSZ)], rows.at[b],
                              ssem[b]).wait()

    ev_load(0, 0, esem[0])
    ev_load(0, 1, esem[1])

    def blk(gg, _):
        first = gg == 0
        not_last = gg < NBLK - 1
        for j in range(WBLK):
            b = j % NRING
            slot = j // EVW
            if j == 0:
                wait_ev(0)
            if j == EVW:
                wait_ev(1)
            if j < NRING:
                @pl.when(~first)
                def _():
                    wait_s(b)
            else:
                wait_s(b)
            for k in range(WSZ // 16):
                sl = pl.ds(16 * k, 16)
                idxv[b, sl] = ev[slot, j - slot * EVW, 0, sl] * 8 \
                    + ev[slot, j - slot * EVW, 1, sl]
                dstv[b, sl] = ev[slot, j - slot * EVW, 2, sl]
            if j == EVW - 1:
                @pl.when(not_last)
                def _():
                    ev_load(gg + 1, 0, esem[0])
            if j == WBLK - 1:
                @pl.when(not_last)
                def _():
                    ev_load(gg + 1, 1, esem[1])
            gather(b)
            pb = (b - 1) % NRING
            if j == 0:
                @pl.when(~first)
                def _():
                    wait_g(pb)
                    scatter(pb)
            else:
                wait_g(pb)
                scatter(pb)
        return 0
    lax.fori_loop(0, NBLK, blk, 0)

    wait_g((WPT - 1) % NRING)
    scatter((WPT - 1) % NRING)
    for b in range(NRING):
        wait_s(b)

    plsc.subcore_barrier()
    pltpu.sync_copy(shared.at[pl.ds(s * ROWS_PER_TILE, ROWS_PER_TILE)],
                    out.at[c, pl.ds(s * ROWS_PER_TILE, ROWS_PER_TILE)])


@functools.cache
def _make_sc_aggr():
    return pl.kernel(
        _sc_aggr_body,
        out_type=jax.ShapeDtypeStruct((NCORE, NP, H), jnp.float32),
        mesh=plsc.VectorSubcoreMesh(core_axis_name="c", subcore_axis_name="s",
                                    num_cores=NCORE, num_subcores=NSUB),
        scratch_types=(
            [
                pltpu.VMEM((2, EVW, 3, WSZ), jnp.int32),
                pltpu.VMEM((NRING, WSZ), jnp.int32),
                pltpu.VMEM((NRING, WSZ), jnp.int32),
                pltpu.VMEM((NRING, WSZ, H), jnp.float32),
                pltpu.VMEM_SHARED((NP, H), jnp.float32),
            ]
            + [pltpu.SemaphoreType.DMA] * 6
        ),
    )


def _sc_aggr(haug, epk):
    return _make_sc_aggr()(haug, epk)


def _aug(h, t_ref):
    return jnp.maximum(h[:, None, :] + t_ref[...][None, :, :], 0.0)


def _onehot(b2d_ref):
    return (b2d_ref[...] == lax.broadcasted_iota(jnp.int32, (1, NG), 1)
            ).astype(jnp.float32)


def _encbuild_body(xf_ref, d_ref, b_ref, t_ref, h_ref, o_ref):
    h = (jnp.dot(xf_ref[...], d_ref[...],
                 preferred_element_type=jnp.float32) + b_ref[...])
    h_ref[...] = h
    o_ref[...] = _aug(h, t_ref)


def _encbuild(xfp, datom, bias, t):
    return pl.pallas_call(
        _encbuild_body,
        grid=(GRID,),
        in_specs=[
            pl.BlockSpec((BN, 9), lambda i: (i, 0)),
            pl.BlockSpec((9, H), lambda i: (0, 0)),
            pl.BlockSpec((1, H), lambda i: (0, 0)),
            pl.BlockSpec((8, H), lambda i: (0, 0)),
        ],
        out_specs=[
            pl.BlockSpec((BN, H), lambda i: (i, 0)),
            pl.BlockSpec((BN, 8, H), lambda i: (i, 0, 0)),
        ],
        out_shape=[
            jax.ShapeDtypeStruct((NP, H), jnp.float32),
            jax.ShapeDtypeStruct((NP, 8, H), jnp.float32),
        ],
    )(xfp, datom, bias, t)


def _mlp(h2, w1_ref, b1_ref, w2_ref, b2_ref):
    m = jnp.maximum(jnp.dot(h2, w1_ref[...],
                            preferred_element_type=jnp.float32)
                    + b1_ref[...], 0.0)
    return (jnp.dot(m, w2_ref[...], preferred_element_type=jnp.float32)
            + b2_ref[...])


def _build_vn_body(h_ref, b2d_ref, vx_ref, t_ref, hc_ref, o_ref):
    oh = _onehot(b2d_ref)
    hc = h_ref[...] + jnp.dot(oh, vx_ref[...],
                              preferred_element_type=jnp.float32)
    hc_ref[...] = hc
    o_ref[...] = _aug(hc, t_ref)


def _build_vn(h, b2d, vx, t):
    return pl.pallas_call(
        _build_vn_body,
        grid=(GRID,),
        in_specs=[
            pl.BlockSpec((BN, H), lambda i: (i, 0)),
            pl.BlockSpec((BN, 1), lambda i: (i, 0)),
            pl.BlockSpec((NG, H), lambda i: (0, 0)),
            pl.BlockSpec((8, H), lambda i: (0, 0)),
        ],
        out_specs=[
            pl.BlockSpec((BN, H), lambda i: (i, 0)),
            pl.BlockSpec((BN, 8, H), lambda i: (i, 0, 0)),
        ],
        out_shape=[
            jax.ShapeDtypeStruct((NP, H), jnp.float32),
            jax.ShapeDtypeStruct((NP, 8, H), jnp.float32),
        ],
    )(h, b2d, vx, t)


_WSPECS = [
    pl.BlockSpec((H, H2), lambda i: (0, 0)),
    pl.BlockSpec((1, H2), lambda i: (0, 0)),
    pl.BlockSpec((H2, H), lambda i: (0, 0)),
    pl.BlockSpec((1, H), lambda i: (0, 0)),
    pl.BlockSpec((1, 1), lambda i: (0, 0)),
]


def _convpool_body(h_ref, p_ref, w1_ref, b1_ref, w2_ref, b2_ref, e_ref,
                   b2d_ref, h_out, pool_out, acc):
    i = pl.program_id(0)
    h2 = h_ref[...] * e_ref[0, 0] + p_ref[0] + p_ref[1]
    hn = jnp.maximum(_mlp(h2, w1_ref, b1_ref, w2_ref, b2_ref), 0.0)
    h_out[...] = hn
    part = lax.dot_general(_onehot(b2d_ref), hn, (((0,), (0,)), ((), ())),
                           preferred_element_type=jnp.float32)

    @pl.when(i == 0)
    def _():
        acc[...] = part

    @pl.when(i > 0)
    def _():
        acc[...] = acc[...] + part

    @pl.when(i == GRID - 1)
    def _():
        pool_out[...] = acc[...]


def _convpool(h, p, w1f, b1f, w2f, b2f, eps, b2d):
    return pl.pallas_call(
        _convpool_body,
        grid=(GRID,),
        in_specs=[
            pl.BlockSpec((BN, H), lambda i: (i, 0)),
            pl.BlockSpec((NCORE, BN, H), lambda i: (0, i, 0)),
        ] + _WSPECS + [pl.BlockSpec((BN, 1), lambda i: (i, 0))],
        out_specs=[
            pl.BlockSpec((BN, H), lambda i: (i, 0)),
            pl.BlockSpec((NG, H), lambda i: (0, 0)),
        ],
        out_shape=[
            jax.ShapeDtypeStruct((NP, H), jnp.float32),
            jax.ShapeDtypeStruct((NG, H), jnp.float32),
        ],
        scratch_shapes=[pltpu.VMEM((NG, H), jnp.float32)],
    )(h, p, w1f, b1f, w2f, b2f, eps, b2d)


def _vnmlp_body(pool_ref, vn0_ref, w1_ref, b1_ref, w2_ref, b2_ref, o_ref):
    row = pool_ref[...] + vn0_ref[...]
    t = jnp.maximum(jnp.dot(row, w1_ref[...],
                            preferred_element_type=jnp.float32)
                    + b1_ref[...], 0.0)
    o_ref[...] = jnp.maximum(jnp.dot(t, w2_ref[...],
                                     preferred_element_type=jnp.float32)
                             + b2_ref[...], 0.0)


def _vnmlp(pooled, vn0, vw1f, vb1f, vw2f, vb2f):
    return pl.pallas_call(
        _vnmlp_body,
        out_shape=jax.ShapeDtypeStruct((NG, H), jnp.float32),
    )(pooled, vn0, vw1f, vb1f, vw2f, vb2f)


def _convbuild_body(h_ref, p_ref, w1_ref, b1_ref, w2_ref, b2_ref, e_ref,
                    t_ref, h_out, aug_out):
    h2 = h_ref[...] * e_ref[0, 0] + p_ref[0] + p_ref[1]
    hn = jnp.maximum(_mlp(h2, w1_ref, b1_ref, w2_ref, b2_ref), 0.0)
    h_out[...] = hn
    aug_out[...] = _aug(hn, t_ref)


def _convbuild(h, p, w1f, b1f, w2f, b2f, eps, t):
    return pl.pallas_call(
        _convbuild_body,
        grid=(GRID,),
        in_specs=[
            pl.BlockSpec((BN, H), lambda i: (i, 0)),
            pl.BlockSpec((NCORE, BN, H), lambda i: (0, i, 0)),
        ] + _WSPECS + [pl.BlockSpec((8, H), lambda i: (0, 0))],
        out_specs=[
            pl.BlockSpec((BN, H), lambda i: (i, 0)),
            pl.BlockSpec((BN, 8, H), lambda i: (i, 0, 0)),
        ],
        out_shape=[
            jax.ShapeDtypeStruct((NP, H), jnp.float32),
            jax.ShapeDtypeStruct((NP, 8, H), jnp.float32),
        ],
    )(h, p, w1f, b1f, w2f, b2f, eps, t)


def _convfinal_body(h_ref, p_ref, w1_ref, b1_ref, w2_ref, b2_ref, e_ref,
                    b2d_ref, w_ref, b_ref, o_ref, acc, cnt):
    i = pl.program_id(0)
    h2 = h_ref[...] * e_ref[0, 0] + p_ref[0] + p_ref[1]
    hn = _mlp(h2, w1_ref, b1_ref, w2_ref, b2_ref)
    oh = _onehot(b2d_ref)
    part = lax.dot_general(oh, hn, (((0,), (0,)), ((), ())),
                           preferred_element_type=jnp.float32)
    ones = jnp.ones((BN, 1), jnp.float32)
    cpart = lax.dot_general(oh, ones, (((0,), (0,)), ((), ())),
                            preferred_element_type=jnp.float32)

    @pl.when(i == 0)
    def _():
        acc[...] = part
        cnt[...] = cpart

    @pl.when(i > 0)
    def _():
        acc[...] = acc[...] + part
        cnt[...] = cnt[...] + cpart

    @pl.when(i == GRID - 1)
    def _():
        hg = acc[...] / jnp.maximum(cnt[...], 1.0)
        o_ref[...] = (jnp.dot(hg, w_ref[...],
                              preferred_element_type=jnp.float32)
                      + b_ref[...])


def _convfinal(h, p, w1f, b1f, w2f, b2f, eps, b2d, out_w, out_b):
    return pl.pallas_call(
        _convfinal_body,
        grid=(GRID,),
        in_specs=[
            pl.BlockSpec((BN, H), lambda i: (i, 0)),
            pl.BlockSpec((NCORE, BN, H), lambda i: (0, i, 0)),
        ] + _WSPECS + [
            pl.BlockSpec((BN, 1), lambda i: (i, 0)),
            pl.BlockSpec((H, H), lambda i: (0, 0)),
            pl.BlockSpec((1, H), lambda i: (0, 0)),
        ],
        out_specs=pl.BlockSpec((NG, H), lambda i: (0, 0)),
        out_shape=jax.ShapeDtypeStruct((NG, H), jnp.float32),
        scratch_shapes=[pltpu.VMEM((NG, H), jnp.float32),
                        pltpu.VMEM((NG, 1), jnp.float32)],
    )(h, p, w1f, b1f, w2f, b2f, eps, b2d, out_w, out_b)


def kernel(x, edge_index, ex, batch, atom_emb, bond_emb, conv_eps, conv_w1,
           conv_b1, conv_bng, conv_bnb, conv_w2, conv_b2, bn_g, bn_b, vn_emb,
           vn_w1, vn_b1, vn_bn1g, vn_bn1b, vn_w2, vn_b2, vn_bn2g, vn_bn2b,
           out_w, out_b):
    f32 = jnp.float32
    rs = 1.0 / jnp.sqrt(1.0 + 1e-5)

    datom = atom_emb[:, 1, :] - atom_emb[:, 0, :]
    encb = (atom_emb[:, 0, :].sum(0) + vn_emb[0])[None, :]
    combos = jnp.array([[c & 1, (c >> 1) & 1, (c >> 2) & 1]
                        for c in range(8)], f32)
    dbond = bond_emb[:, :, 1, :] - bond_emb[:, :, 0, :]
    cbond = bond_emb[:, :, 0, :].sum(1)
    tables = cbond[:, None, :] + jnp.einsum("cj,ljh->lch", combos, dbond)

    s1 = conv_bng * rs
    w1f = conv_w1 * s1[:, None, :]
    b1f = conv_b1 * s1 + conv_bnb
    s2 = jnp.concatenate([bn_g * rs, jnp.ones((1, H), f32)], 0)
    badd = jnp.concatenate([bn_b, jnp.zeros((1, H), f32)], 0)
    w2f = conv_w2 * s2[:, None, :]
    b2f = conv_b2 * s2 + badd
    sv1 = vn_bn1g[0] * rs
    vw1f = vn_w1[0] * sv1[None, :]
    vb1f = (vn_b1[0] * sv1 + vn_bn1b[0])[None, :]
    sv2 = vn_bn2g[0] * rs
    vw2f = vn_w2[0] * sv2[None, :]
    vb2f = (vn_b2[0] * sv2 + vn_bn2b[0])[None, :]
    eps1 = (1.0 + conv_eps).astype(f32)

    xfp = jnp.zeros((NP, 9), f32).at[:N].set(x.astype(f32))
    batchpad = jnp.concatenate(
        [batch.astype(jnp.int32), jnp.full((NP - N,), -1, jnp.int32)])
    b2d = batchpad.reshape(NP, 1)

    src = edge_index[0].astype(jnp.int32)
    dst = edge_index[1].astype(jnp.int32)
    code = (ex[:, 0] + 2 * ex[:, 1] + 4 * ex[:, 2]).astype(jnp.int32)
    padsrc = (jnp.arange(PAD, dtype=jnp.int32) * 89) % N
    paddst = N + (jnp.arange(PAD, dtype=jnp.int32) % (NP - N))
    padcode = jnp.zeros((PAD,), jnp.int32)

    def tile_pack(a, padvals):
        a2 = a.reshape(TILES, PTE)
        p = jnp.broadcast_to(padvals, (TILES, PAD))
        return jnp.concatenate([a2, p], axis=1).reshape(TILES * WPT, WSZ)

    epk = jnp.stack([tile_pack(src, padsrc), tile_pack(code, padcode),
                     tile_pack(dst, paddst)], axis=1)

    h_a, aug0 = _encbuild(xfp, datom, encb, tables[0])
    p = _sc_aggr(aug0.reshape(NP * 8, H), epk)
    h_b, pooled = _convpool(h_a, p, w1f[0], b1f[0][None], w2f[0],
                            b2f[0][None], eps1[0].reshape(1, 1), b2d)
    vx = _vnmlp(pooled, vn_emb[0][None], vw1f, vb1f, vw2f, vb2f)
    h_c, aug1 = _build_vn(h_b, b2d, vx, tables[1])
    p = _sc_aggr(aug1.reshape(NP * 8, H), epk)
    h_d, aug2 = _convbuild(h_c, p, w1f[1], b1f[1][None], w2f[1],
                           b2f[1][None], eps1[1].reshape(1, 1), tables[2])
    p = _sc_aggr(aug2.reshape(NP * 8, H), epk)
    return _convfinal(h_d, p, w1f[2], b1f[2][None], w2f[2], b2f[2][None],
                      eps1[2].reshape(1, 1), b2d, out_w, out_b[None, :])


# speedup vs baseline: 21.2480x; 1.0224x over previous
import functools

import jax
import jax.numpy as jnp
from jax import lax
from jax.experimental import pallas as pl
from jax.experimental.pallas import tpu as pltpu
from jax.experimental.pallas import tpu_sc as plsc

N = 10000
NP = 10240
E = 320000
H = 128
H2 = 256
NG = 128
L = 3

BN = 2048
GRID = NP // BN

NCORE = 2
NSUB = 16
TILES = NCORE * NSUB
PTE = E // TILES
WSZ = 128
WPT = 80
PAD = WPT * WSZ - PTE
ROWS_PER_TILE = NP // NSUB
NRING = 2
EVW = 4
WBLK = 8
NBLK = WPT // WBLK


def _sc_aggr_body(haug, epk, out, ev, idxv, dstv, rows, shared,
                  gs0, gs1, ss0, ss1, es0, es1):
    c = lax.axis_index("c")
    s = lax.axis_index("s")
    wid = c * NSUB + s
    base = wid * WPT
    gsem = (gs0, gs1)
    ssem = (ss0, ss1)
    esem = (es0, es1)

    def zrow(r, _):
        for k in range(H // 16):
            rows[0, r, pl.ds(16 * k, 16)] = jnp.zeros((16,), jnp.float32)
        return 0
    lax.fori_loop(0, WSZ, zrow, 0)

    for j in range(ROWS_PER_TILE // WSZ):
        pltpu.async_copy(rows.at[0],
                         shared.at[pl.ds(s * ROWS_PER_TILE + j * WSZ, WSZ)],
                         ss0)
    for j in range(ROWS_PER_TILE // WSZ):
        pltpu.make_async_copy(haug.at[pl.ds(0, WSZ)], rows.at[0], ss0).wait()
    plsc.subcore_barrier()

    def ev_load(blk, slot, sem):
        return pltpu.async_copy(
            epk.at[pl.ds(base + blk * WBLK + slot * EVW, EVW)],
            ev.at[slot], sem)

    def gather(b):
        return pltpu.async_copy(haug.at[idxv.at[b]], rows.at[b], gsem[b])

    def scatter(b):
        return pltpu.async_copy(rows.at[b], shared.at[dstv.at[b]], ssem[b],
                                add=True)

    def wait_ev(slot):
        pltpu.make_async_copy(epk.at[pl.ds(base, EVW)], ev.at[slot],
                              esem[slot]).wait()

    def wait_g(b):
        pltpu.make_async_copy(haug.at[pl.ds(0, WSZ)], rows.at[b],
                              gsem[b]).wait()

    def wait_s(b):
        pltpu.make_async_copy(haug.at[pl.ds(0, WSZ)], rows.at[b],
                              ssem[b]).wait()

    ev_load(0, 0, esem[0])
    ev_load(0, 1, esem[1])

    def blk(gg, _):
        first = gg == 0
        not_last = gg < NBLK - 1
        for j in range(WBLK):
            b = j % NRING
            slot = j // EVW
            if j == 0:
                wait_ev(0)
            if j == EVW:
                wait_ev(1)
            if j < NRING:
                @pl.when(~first)
                def _():
                    wait_s(b)
            else:
                wait_s(b)
            for k in range(WSZ // 16):
                sl = pl.ds(16 * k, 16)
                idxv[b, sl] = ev[slot, j - slot * EVW, 0, sl] * 8 \
                    + ev[slot, j - slot * EVW, 1, sl]
                dstv[b, sl] = ev[slot, j - slot * EVW, 2, sl]
            if j == EVW - 1:
                @pl.when(not_last)
                def _():
                    ev_load(gg + 1, 0, esem[0])
            if j == WBLK - 1:
                @pl.when(not_last)
                def _():
                    ev_load(gg + 1, 1, esem[1])
            gather(b)
            pb = (b - 1) % NRING
            if j == 0:
                @pl.when(~first)
                def _():
                    wait_g(pb)
                    scatter(pb)
            else:
                wait_g(pb)
                scatter(pb)
        return 0
    lax.fori_loop(0, NBLK, blk, 0)

    wait_g((WPT - 1) % NRING)
    scatter((WPT - 1) % NRING)
    for b in range(NRING):
        wait_s(b)

    plsc.subcore_barrier()
    pltpu.sync_copy(shared.at[pl.ds(s * ROWS_PER_TILE, ROWS_PER_TILE)],
                    out.at[c, pl.ds(s * ROWS_PER_TILE, ROWS_PER_TILE)])


@functools.cache
def _make_sc_aggr():
    return pl.kernel(
        _sc_aggr_body,
        out_type=jax.ShapeDtypeStruct((NCORE, NP, H), jnp.float32),
        mesh=plsc.VectorSubcoreMesh(core_axis_name="c", subcore_axis_name="s",
                                    num_cores=NCORE, num_subcores=NSUB),
        scratch_types=(
            [
                pltpu.VMEM((2, EVW, 3, WSZ), jnp.int32),
                pltpu.VMEM((NRING, WSZ), jnp.int32),
                pltpu.VMEM((NRING, WSZ), jnp.int32),
                pltpu.VMEM((NRING, WSZ, H), jnp.float32),
                pltpu.VMEM_SHARED((NP, H), jnp.float32),
            ]
            + [pltpu.SemaphoreType.DMA] * 6
        ),
    )


def _sc_aggr(haug, epk):
    return _make_sc_aggr()(haug, epk)


def _aug(h, t_ref):
    return jnp.maximum(h[:, None, :] + t_ref[...][None, :, :], 0.0)


def _onehot(b2d_ref):
    return (b2d_ref[...] == lax.broadcasted_iota(jnp.int32, (1, NG), 1)
            ).astype(jnp.float32)


def _encbuild_body(xf_ref, d_ref, b_ref, t_ref, h_ref, o_ref):
    h = (jnp.dot(xf_ref[...], d_ref[...],
                 preferred_element_type=jnp.float32) + b_ref[...])
    h_ref[...] = h
    o_ref[...] = _aug(h, t_ref)


def _encbuild(xfp, datom, bias, t):
    return pl.pallas_call(
        _encbuild_body,
        grid=(GRID,),
        in_specs=[
            pl.BlockSpec((BN, 9), lambda i: (i, 0)),
            pl.BlockSpec((9, H), lambda i: (0, 0)),
            pl.BlockSpec((1, H), lambda i: (0, 0)),
            pl.BlockSpec((8, H), lambda i: (0, 0)),
        ],
        out_specs=[
            pl.BlockSpec((BN, H), lambda i: (i, 0)),
            pl.BlockSpec((BN, 8, H), lambda i: (i, 0, 0)),
        ],
        out_shape=[
            jax.ShapeDtypeStruct((NP, H), jnp.float32),
            jax.ShapeDtypeStruct((NP, 8, H), jnp.float32),
        ],
    )(xfp, datom, bias, t)


def _mlp(h2, w1_ref, b1_ref, w2_ref, b2_ref):
    m = jnp.maximum(jnp.dot(h2, w1_ref[...],
                            preferred_element_type=jnp.float32)
                    + b1_ref[...], 0.0)
    return (jnp.dot(m, w2_ref[...], preferred_element_type=jnp.float32)
            + b2_ref[...])


def _build_vn_body(h_ref, b2d_ref, pool_ref, vn0_ref, w1_ref, b1_ref,
                   w2_ref, b2_ref, t_ref, hc_ref, o_ref, vxs):
    @pl.when(pl.program_id(0) == 0)
    def _():
        row = pool_ref[...] + vn0_ref[...]
        t = jnp.maximum(jnp.dot(row, w1_ref[...],
                                preferred_element_type=jnp.float32)
                        + b1_ref[...], 0.0)
        vxs[...] = jnp.maximum(jnp.dot(t, w2_ref[...],
                                       preferred_element_type=jnp.float32)
                               + b2_ref[...], 0.0)

    oh = _onehot(b2d_ref)
    hc = h_ref[...] + jnp.dot(oh, vxs[...],
                              preferred_element_type=jnp.float32)
    hc_ref[...] = hc
    o_ref[...] = _aug(hc, t_ref)


def _build_vn(h, b2d, pooled, vn0, vw1f, vb1f, vw2f, vb2f, t):
    return pl.pallas_call(
        _build_vn_body,
        grid=(GRID,),
        in_specs=[
            pl.BlockSpec((BN, H), lambda i: (i, 0)),
            pl.BlockSpec((BN, 1), lambda i: (i, 0)),
            pl.BlockSpec((NG, H), lambda i: (0, 0)),
            pl.BlockSpec((1, H), lambda i: (0, 0)),
            pl.BlockSpec((H, H2), lambda i: (0, 0)),
            pl.BlockSpec((1, H2), lambda i: (0, 0)),
            pl.BlockSpec((H2, H), lambda i: (0, 0)),
            pl.BlockSpec((1, H), lambda i: (0, 0)),
            pl.BlockSpec((8, H), lambda i: (0, 0)),
        ],
        out_specs=[
            pl.BlockSpec((BN, H), lambda i: (i, 0)),
            pl.BlockSpec((BN, 8, H), lambda i: (i, 0, 0)),
        ],
        out_shape=[
            jax.ShapeDtypeStruct((NP, H), jnp.float32),
            jax.ShapeDtypeStruct((NP, 8, H), jnp.float32),
        ],
        scratch_shapes=[pltpu.VMEM((NG, H), jnp.float32)],
    )(h, b2d, pooled, vn0, vw1f, vb1f, vw2f, vb2f, t)


_WSPECS = [
    pl.BlockSpec((H, H2), lambda i: (0, 0)),
    pl.BlockSpec((1, H2), lambda i: (0, 0)),
    pl.BlockSpec((H2, H), lambda i: (0, 0)),
    pl.BlockSpec((1, H), lambda i: (0, 0)),
    pl.BlockSpec((1, 1), lambda i: (0, 0)),
]


def _convpool_body(h_ref, p_ref, w1_ref, b1_ref, w2_ref, b2_ref, e_ref,
                   b2d_ref, h_out, pool_out, acc):
    i = pl.program_id(0)
    h2 = h_ref[...] * e_ref[0, 0] + p_ref[0] + p_ref[1]
    hn = jnp.maximum(_mlp(h2, w1_ref, b1_ref, w2_ref, b2_ref), 0.0)
    h_out[...] = hn
    part = lax.dot_general(_onehot(b2d_ref), hn, (((0,), (0,)), ((), ())),
                           preferred_element_type=jnp.float32)

    @pl.when(i == 0)
    def _():
        acc[...] = part

    @pl.when(i > 0)
    def _():
        acc[...] = acc[...] + part

    @pl.when(i == GRID - 1)
    def _():
        pool_out[...] = acc[...]


def _convpool(h, p, w1f, b1f, w2f, b2f, eps, b2d):
    return pl.pallas_call(
        _convpool_body,
        grid=(GRID,),
        in_specs=[
            pl.BlockSpec((BN, H), lambda i: (i, 0)),
            pl.BlockSpec((NCORE, BN, H), lambda i: (0, i, 0)),
        ] + _WSPECS + [pl.BlockSpec((BN, 1), lambda i: (i, 0))],
        out_specs=[
            pl.BlockSpec((BN, H), lambda i: (i, 0)),
            pl.BlockSpec((NG, H), lambda i: (0, 0)),
        ],
        out_shape=[
            jax.ShapeDtypeStruct((NP, H), jnp.float32),
            jax.ShapeDtypeStruct((NG, H), jnp.float32),
        ],
        scratch_shapes=[pltpu.VMEM((NG, H), jnp.float32)],
    )(h, p, w1f, b1f, w2f, b2f, eps, b2d)


def _convbuild_body(h_ref, p_ref, w1_ref, b1_ref, w2_ref, b2_ref, e_ref,
                    t_ref, h_out, aug_out):
    h2 = h_ref[...] * e_ref[0, 0] + p_ref[0] + p_ref[1]
    hn = jnp.maximum(_mlp(h2, w1_ref, b1_ref, w2_ref, b2_ref), 0.0)
    h_out[...] = hn
    aug_out[...] = _aug(hn, t_ref)


def _convbuild(h, p, w1f, b1f, w2f, b2f, eps, t):
    return pl.pallas_call(
        _convbuild_body,
        grid=(GRID,),
        in_specs=[
            pl.BlockSpec((BN, H), lambda i: (i, 0)),
            pl.BlockSpec((NCORE, BN, H), lambda i: (0, i, 0)),
        ] + _WSPECS + [pl.BlockSpec((8, H), lambda i: (0, 0))],
        out_specs=[
            pl.BlockSpec((BN, H), lambda i: (i, 0)),
            pl.BlockSpec((BN, 8, H), lambda i: (i, 0, 0)),
        ],
        out_shape=[
            jax.ShapeDtypeStruct((NP, H), jnp.float32),
            jax.ShapeDtypeStruct((NP, 8, H), jnp.float32),
        ],
    )(h, p, w1f, b1f, w2f, b2f, eps, t)


def _convfinal_body(h_ref, p_ref, w1_ref, b1_ref, w2_ref, b2_ref, e_ref,
                    b2d_ref, w_ref, b_ref, o_ref, acc, cnt):
    i = pl.program_id(0)
    h2 = h_ref[...] * e_ref[0, 0] + p_ref[0] + p_ref[1]
    hn = _mlp(h2, w1_ref, b1_ref, w2_ref, b2_ref)
    oh = _onehot(b2d_ref)
    part = lax.dot_general(oh, hn, (((0,), (0,)), ((), ())),
                           preferred_element_type=jnp.float32)
    ones = jnp.ones((BN, 1), jnp.float32)
    cpart = lax.dot_general(oh, ones, (((0,), (0,)), ((), ())),
                            preferred_element_type=jnp.float32)

    @pl.when(i == 0)
    def _():
        acc[...] = part
        cnt[...] = cpart

    @pl.when(i > 0)
    def _():
        acc[...] = acc[...] + part
        cnt[...] = cnt[...] + cpart

    @pl.when(i == GRID - 1)
    def _():
        hg = acc[...] / jnp.maximum(cnt[...], 1.0)
        o_ref[...] = (jnp.dot(hg, w_ref[...],
                              preferred_element_type=jnp.float32)
                      + b_ref[...])


def _convfinal(h, p, w1f, b1f, w2f, b2f, eps, b2d, out_w, out_b):
    return pl.pallas_call(
        _convfinal_body,
        grid=(GRID,),
        in_specs=[
            pl.BlockSpec((BN, H), lambda i: (i, 0)),
            pl.BlockSpec((NCORE, BN, H), lambda i: (0, i, 0)),
        ] + _WSPECS + [
            pl.BlockSpec((BN, 1), lambda i: (i, 0)),
            pl.BlockSpec((H, H), lambda i: (0, 0)),
            pl.BlockSpec((1, H), lambda i: (0, 0)),
        ],
        out_specs=pl.BlockSpec((NG, H), lambda i: (0, 0)),
        out_shape=jax.ShapeDtypeStruct((NG, H), jnp.float32),
        scratch_shapes=[pltpu.VMEM((NG, H), jnp.float32),
                        pltpu.VMEM((NG, 1), jnp.float32)],
    )(h, p, w1f, b1f, w2f, b2f, eps, b2d, out_w, out_b)


def kernel(x, edge_index, ex, batch, atom_emb, bond_emb, conv_eps, conv_w1,
           conv_b1, conv_bng, conv_bnb, conv_w2, conv_b2, bn_g, bn_b, vn_emb,
           vn_w1, vn_b1, vn_bn1g, vn_bn1b, vn_w2, vn_b2, vn_bn2g, vn_bn2b,
           out_w, out_b):
    f32 = jnp.float32
    rs = 1.0 / jnp.sqrt(1.0 + 1e-5)

    datom = atom_emb[:, 1, :] - atom_emb[:, 0, :]
    encb = (atom_emb[:, 0, :].sum(0) + vn_emb[0])[None, :]
    combos = jnp.array([[c & 1, (c >> 1) & 1, (c >> 2) & 1]
                        for c in range(8)], f32)
    dbond = bond_emb[:, :, 1, :] - bond_emb[:, :, 0, :]
    cbond = bond_emb[:, :, 0, :].sum(1)
    tables = cbond[:, None, :] + jnp.einsum("cj,ljh->lch", combos, dbond)

    s1 = conv_bng * rs
    w1f = conv_w1 * s1[:, None, :]
    b1f = conv_b1 * s1 + conv_bnb
    s2 = jnp.concatenate([bn_g * rs, jnp.ones((1, H), f32)], 0)
    badd = jnp.concatenate([bn_b, jnp.zeros((1, H), f32)], 0)
    w2f = conv_w2 * s2[:, None, :]
    b2f = conv_b2 * s2 + badd
    sv1 = vn_bn1g[0] * rs
    vw1f = vn_w1[0] * sv1[None, :]
    vb1f = (vn_b1[0] * sv1 + vn_bn1b[0])[None, :]
    sv2 = vn_bn2g[0] * rs
    vw2f = vn_w2[0] * sv2[None, :]
    vb2f = (vn_b2[0] * sv2 + vn_bn2b[0])[None, :]
    eps1 = (1.0 + conv_eps).astype(f32)

    xfp = jnp.zeros((NP, 9), f32).at[:N].set(x.astype(f32))
    batchpad = jnp.concatenate(
        [batch.astype(jnp.int32), jnp.full((NP - N,), -1, jnp.int32)])
    b2d = batchpad.reshape(NP, 1)

    src = edge_index[0].astype(jnp.int32)
    dst = edge_index[1].astype(jnp.int32)
    code = (ex[:, 0] + 2 * ex[:, 1] + 4 * ex[:, 2]).astype(jnp.int32)
    padsrc = (jnp.arange(PAD, dtype=jnp.int32) * 89) % N
    paddst = N + (jnp.arange(PAD, dtype=jnp.int32) % (NP - N))
    padcode = jnp.zeros((PAD,), jnp.int32)

    def tile_pack(a, padvals):
        a2 = a.reshape(TILES, PTE)
        p = jnp.broadcast_to(padvals, (TILES, PAD))
        return jnp.concatenate([a2, p], axis=1).reshape(TILES * WPT, WSZ)

    epk = jnp.stack([tile_pack(src, padsrc), tile_pack(code, padcode),
                     tile_pack(dst, paddst)], axis=1)

    h_a, aug0 = _encbuild(xfp, datom, encb, tables[0])
    p = _sc_aggr(aug0.reshape(NP * 8, H), epk)
    h_b, pooled = _convpool(h_a, p, w1f[0], b1f[0][None], w2f[0],
                            b2f[0][None], eps1[0].reshape(1, 1), b2d)
    h_c, aug1 = _build_vn(h_b, b2d, pooled, vn_emb[0][None], vw1f, vb1f,
                          vw2f, vb2f, tables[1])
    p = _sc_aggr(aug1.reshape(NP * 8, H), epk)
    h_d, aug2 = _convbuild(h_c, p, w1f[1], b1f[1][None], w2f[1],
                           b2f[1][None], eps1[1].reshape(1, 1), tables[2])
    p = _sc_aggr(aug2.reshape(NP * 8, H), epk)
    return _convfinal(h_d, p, w1f[2], b1f[2][None], w2f[2], b2f[2][None],
                      eps1[2].reshape(1, 1), b2d, out_w, out_b[None, :])


# speedup vs baseline: 21.4068x; 1.0075x over previous
import functools

import jax
import jax.numpy as jnp
from jax import lax
from jax.experimental import pallas as pl
from jax.experimental.pallas import tpu as pltpu
from jax.experimental.pallas import tpu_sc as plsc

N = 10000
NP = 10240
E = 320000
H = 128
H2 = 256
NG = 128
L = 3

BN = 2048
GRID = NP // BN

NCORE = 2
NSUB = 16
TILES = NCORE * NSUB
PTE = E // TILES
WSZ = 128
WPT = 80
PAD = WPT * WSZ - PTE
ROWS_PER_TILE = NP // NSUB
NRING = 2
EVW = 4
WBLK = 8
NBLK = WPT // WBLK


def _sc_aggr_body(haug, epk, out, ev, idxv, dstv, rows, shared,
                  gs0, gs1, ss0, ss1, es0, es1):
    c = lax.axis_index("c")
    s = lax.axis_index("s")
    wid = c * NSUB + s
    base = wid * WPT
    gsem = (gs0, gs1)
    ssem = (ss0, ss1)
    esem = (es0, es1)

    def zrow(r, _):
        for k in range(H // 16):
            rows[0, r, pl.ds(16 * k, 16)] = jnp.zeros((16,), jnp.float32)
        return 0
    lax.fori_loop(0, WSZ, zrow, 0)

    for j in range(ROWS_PER_TILE // WSZ):
        pltpu.async_copy(rows.at[0],
                         shared.at[pl.ds(s * ROWS_PER_TILE + j * WSZ, WSZ)],
                         ss0)
    for j in range(ROWS_PER_TILE // WSZ):
        pltpu.make_async_copy(haug.at[pl.ds(0, WSZ)], rows.at[0], ss0).wait()
    plsc.subcore_barrier()

    def ev_load(blk, slot, sem):
        return pltpu.async_copy(
            epk.at[pl.ds(base + blk * WBLK + slot * EVW, EVW)],
            ev.at[slot], sem)

    def gather(b):
        return pltpu.async_copy(haug.at[idxv.at[b]], rows.at[b], gsem[b])

    def scatter(b):
        return pltpu.async_copy(rows.at[b], shared.at[dstv.at[b]], ssem[b],
                                add=True)

    def wait_ev(slot):
        pltpu.make_async_copy(epk.at[pl.ds(base, EVW)], ev.at[slot],
                              esem[slot]).wait()

    def wait_g(b):
        pltpu.make_async_copy(haug.at[pl.ds(0, WSZ)], rows.at[b],
                              gsem[b]).wait()

    def wait_s(b):
        pltpu.make_async_copy(haug.at[pl.ds(0, WSZ)], rows.at[b],
                              ssem[b]).wait()

    ev_load(0, 0, esem[0])
    ev_load(0, 1, esem[1])

    def blk(gg, _):
        first = gg == 0
        not_last = gg < NBLK - 1
        for j in range(WBLK):
            b = j % NRING
            slot = j // EVW
            if j == 0:
                wait_ev(0)
            if j == EVW:
                wait_ev(1)
            if j < NRING:
                @pl.when(~first)
                def _():
                    wait_s(b)
            else:
                wait_s(b)
            for k in range(WSZ // 16):
                sl = pl.ds(16 * k, 16)
                idxv[b, sl] = ev[slot, j - slot * EVW, 0, sl] * 8 \
                    + ev[slot, j - slot * EVW, 1, sl]
                dstv[b, sl] = ev[slot, j - slot * EVW, 2, sl]
            if j == EVW - 1:
                @pl.when(not_last)
                def _():
                    ev_load(gg + 1, 0, esem[0])
            if j == WBLK - 1:
                @pl.when(not_last)
                def _():
                    ev_load(gg + 1, 1, esem[1])
            gather(b)
            pb = (b - 1) % NRING
            if j == 0:
                @pl.when(~first)
                def _():
                    wait_g(pb)
                    scatter(pb)
            else:
                wait_g(pb)
                scatter(pb)
        return 0
    lax.fori_loop(0, NBLK, blk, 0)

    wait_g((WPT - 1) % NRING)
    scatter((WPT - 1) % NRING)
    for b in range(NRING):
        wait_s(b)

    plsc.subcore_barrier()
    pltpu.sync_copy(shared.at[pl.ds(s * ROWS_PER_TILE, ROWS_PER_TILE)],
                    out.at[c, pl.ds(s * ROWS_PER_TILE, ROWS_PER_TILE)])


@functools.cache
def _make_sc_aggr():
    return pl.kernel(
        _sc_aggr_body,
        out_type=jax.ShapeDtypeStruct((NCORE, NP, H), jnp.float32),
        mesh=plsc.VectorSubcoreMesh(core_axis_name="c", subcore_axis_name="s",
                                    num_cores=NCORE, num_subcores=NSUB),
        scratch_types=(
            [
                pltpu.VMEM((2, EVW, 3, WSZ), jnp.int32),
                pltpu.VMEM((NRING, WSZ), jnp.int32),
                pltpu.VMEM((NRING, WSZ), jnp.int32),
                pltpu.VMEM((NRING, WSZ, H), jnp.float32),
                pltpu.VMEM_SHARED((NP, H), jnp.float32),
            ]
            + [pltpu.SemaphoreType.DMA] * 6
        ),
    )


def _sc_aggr(haug, epk):
    return _make_sc_aggr()(haug, epk)


def _aug(h, t_ref):
    return jnp.maximum(h[:, None, :] + t_ref[...][None, :, :], 0.0)


def _onehot(b2d_ref):
    return (b2d_ref[...] == lax.broadcasted_iota(jnp.int32, (1, NG), 1)
            ).astype(jnp.float32)


def _encbuild_body(xf_ref, d_ref, b_ref, t_ref, h_ref, o_ref):
    h = (jnp.dot(xf_ref[...], d_ref[...],
                 preferred_element_type=jnp.float32) + b_ref[...])
    h_ref[...] = h
    o_ref[...] = _aug(h, t_ref)


def _encbuild(xfp, datom, bias, t):
    return pl.pallas_call(
        _encbuild_body,
        grid=(GRID,),
        in_specs=[
            pl.BlockSpec((BN, 9), lambda i: (i, 0)),
            pl.BlockSpec((9, H), lambda i: (0, 0)),
            pl.BlockSpec((1, H), lambda i: (0, 0)),
            pl.BlockSpec((8, H), lambda i: (0, 0)),
        ],
        out_specs=[
            pl.BlockSpec((BN, H), lambda i: (i, 0)),
            pl.BlockSpec((BN, 8, H), lambda i: (i, 0, 0)),
        ],
        out_shape=[
            jax.ShapeDtypeStruct((NP, H), jnp.float32),
            jax.ShapeDtypeStruct((NP, 8, H), jnp.float32),
        ],
    )(xfp, datom, bias, t)


def _mlp(h2, w1_ref, b1_ref, w2_ref, b2_ref):
    m = jnp.maximum(jnp.dot(h2, w1_ref[...],
                            preferred_element_type=jnp.float32)
                    + b1_ref[...], 0.0)
    return (jnp.dot(m, w2_ref[...], preferred_element_type=jnp.float32)
            + b2_ref[...])


_WSPECS = [
    pl.BlockSpec((H, H2), lambda i: (0, 0)),
    pl.BlockSpec((1, H2), lambda i: (0, 0)),
    pl.BlockSpec((H2, H), lambda i: (0, 0)),
    pl.BlockSpec((1, H), lambda i: (0, 0)),
    pl.BlockSpec((1, 1), lambda i: (0, 0)),
]


def _cpb_body(h_ref, p_ref, w1_ref, b1_ref, w2_ref, b2_ref, e_ref,
              b2d_ref, vn0_ref, vw1_ref, vb1_ref, vw2_ref, vb2_ref, t_ref,
              hc_ref, aug_ref, hbuf, acc, vxs):
    i = pl.program_id(0)

    @pl.when(i < GRID)
    def _():
        h2 = h_ref[...] * e_ref[0, 0] + p_ref[0] + p_ref[1]
        hn = jnp.maximum(_mlp(h2, w1_ref, b1_ref, w2_ref, b2_ref), 0.0)
        hbuf[pl.ds(i * BN, BN), :] = hn
        part = lax.dot_general(_onehot(b2d_ref), hn,
                               (((0,), (0,)), ((), ())),
                               preferred_element_type=jnp.float32)

        @pl.when(i == 0)
        def _():
            acc[...] = part

        @pl.when(i > 0)
        def _():
            acc[...] = acc[...] + part

    @pl.when(i == GRID - 1)
    def _():
        row = acc[...] + vn0_ref[...]
        t = jnp.maximum(jnp.dot(row, vw1_ref[...],
                                preferred_element_type=jnp.float32)
                        + vb1_ref[...], 0.0)
        vxs[...] = jnp.maximum(jnp.dot(t, vw2_ref[...],
                                       preferred_element_type=jnp.float32)
                               + vb2_ref[...], 0.0)

    @pl.when(i >= GRID)
    def _():
        hb = hbuf[pl.ds((i - GRID) * BN, BN), :]
        hc = hb + jnp.dot(_onehot(b2d_ref), vxs[...],
                          preferred_element_type=jnp.float32)
        hc_ref[...] = hc
        aug_ref[...] = _aug(hc, t_ref)


def _convpoolbuild(h, p, w1f, b1f, w2f, b2f, eps, b2d, vn0, vw1f, vb1f,
                   vw2f, vb2f, t):
    ph1 = lambda i: (jnp.where(i < GRID, i, 0), 0)
    both = lambda i: (i % GRID, 0)
    out2 = lambda i: (jnp.maximum(i - GRID, 0), 0)
    return pl.pallas_call(
        _cpb_body,
        grid=(2 * GRID,),
        in_specs=[
            pl.BlockSpec((BN, H), ph1),
            pl.BlockSpec((NCORE, BN, H),
                         lambda i: (0, jnp.where(i < GRID, i, 0), 0)),
        ] + _WSPECS + [
            pl.BlockSpec((BN, 1), both),
            pl.BlockSpec((1, H), lambda i: (0, 0)),
            pl.BlockSpec((H, H2), lambda i: (0, 0)),
            pl.BlockSpec((1, H2), lambda i: (0, 0)),
            pl.BlockSpec((H2, H), lambda i: (0, 0)),
            pl.BlockSpec((1, H), lambda i: (0, 0)),
            pl.BlockSpec((8, H), lambda i: (0, 0)),
        ],
        out_specs=[
            pl.BlockSpec((BN, H), out2),
            pl.BlockSpec((BN, 8, H), lambda i: (jnp.maximum(i - GRID, 0),
                                                0, 0)),
        ],
        out_shape=[
            jax.ShapeDtypeStruct((NP, H), jnp.float32),
            jax.ShapeDtypeStruct((NP, 8, H), jnp.float32),
        ],
        scratch_shapes=[pltpu.VMEM((NP, H), jnp.float32),
                        pltpu.VMEM((NG, H), jnp.float32),
                        pltpu.VMEM((NG, H), jnp.float32)],
    )(h, p, w1f, b1f, w2f, b2f, eps, b2d, vn0, vw1f, vb1f, vw2f, vb2f, t)


def _convbuild_body(h_ref, p_ref, w1_ref, b1_ref, w2_ref, b2_ref, e_ref,
                    t_ref, h_out, aug_out):
    h2 = h_ref[...] * e_ref[0, 0] + p_ref[0] + p_ref[1]
    hn = jnp.maximum(_mlp(h2, w1_ref, b1_ref, w2_ref, b2_ref), 0.0)
    h_out[...] = hn
    aug_out[...] = _aug(hn, t_ref)


def _convbuild(h, p, w1f, b1f, w2f, b2f, eps, t):
    return pl.pallas_call(
        _convbuild_body,
        grid=(GRID,),
        in_specs=[
            pl.BlockSpec((BN, H), lambda i: (i, 0)),
            pl.BlockSpec((NCORE, BN, H), lambda i: (0, i, 0)),
        ] + _WSPECS + [pl.BlockSpec((8, H), lambda i: (0, 0))],
        out_specs=[
            pl.BlockSpec((BN, H), lambda i: (i, 0)),
            pl.BlockSpec((BN, 8, H), lambda i: (i, 0, 0)),
        ],
        out_shape=[
            jax.ShapeDtypeStruct((NP, H), jnp.float32),
            jax.ShapeDtypeStruct((NP, 8, H), jnp.float32),
        ],
    )(h, p, w1f, b1f, w2f, b2f, eps, t)


def _convfinal_body(h_ref, p_ref, w1_ref, b1_ref, w2_ref, b2_ref, e_ref,
                    b2d_ref, w_ref, b_ref, o_ref, acc, cnt):
    i = pl.program_id(0)
    h2 = h_ref[...] * e_ref[0, 0] + p_ref[0] + p_ref[1]
    hn = _mlp(h2, w1_ref, b1_ref, w2_ref, b2_ref)
    oh = _onehot(b2d_ref)
    part = lax.dot_general(oh, hn, (((0,), (0,)), ((), ())),
                           preferred_element_type=jnp.float32)
    ones = jnp.ones((BN, 1), jnp.float32)
    cpart = lax.dot_general(oh, ones, (((0,), (0,)), ((), ())),
                            preferred_element_type=jnp.float32)

    @pl.when(i == 0)
    def _():
        acc[...] = part
        cnt[...] = cpart

    @pl.when(i > 0)
    def _():
        acc[...] = acc[...] + part
        cnt[...] = cnt[...] + cpart

    @pl.when(i == GRID - 1)
    def _():
        hg = acc[...] / jnp.maximum(cnt[...], 1.0)
        o_ref[...] = (jnp.dot(hg, w_ref[...],
                              preferred_element_type=jnp.float32)
                      + b_ref[...])


def _convfinal(h, p, w1f, b1f, w2f, b2f, eps, b2d, out_w, out_b):
    return pl.pallas_call(
        _convfinal_body,
        grid=(GRID,),
        in_specs=[
            pl.BlockSpec((BN, H), lambda i: (i, 0)),
            pl.BlockSpec((NCORE, BN, H), lambda i: (0, i, 0)),
        ] + _WSPECS + [
            pl.BlockSpec((BN, 1), lambda i: (i, 0)),
            pl.BlockSpec((H, H), lambda i: (0, 0)),
            pl.BlockSpec((1, H), lambda i: (0, 0)),
        ],
        out_specs=pl.BlockSpec((NG, H), lambda i: (0, 0)),
        out_shape=jax.ShapeDtypeStruct((NG, H), jnp.float32),
        scratch_shapes=[pltpu.VMEM((NG, H), jnp.float32),
                        pltpu.VMEM((NG, 1), jnp.float32)],
    )(h, p, w1f, b1f, w2f, b2f, eps, b2d, out_w, out_b)


def kernel(x, edge_index, ex, batch, atom_emb, bond_emb, conv_eps, conv_w1,
           conv_b1, conv_bng, conv_bnb, conv_w2, conv_b2, bn_g, bn_b, vn_emb,
           vn_w1, vn_b1, vn_bn1g, vn_bn1b, vn_w2, vn_b2, vn_bn2g, vn_bn2b,
           out_w, out_b):
    f32 = jnp.float32
    rs = 1.0 / jnp.sqrt(1.0 + 1e-5)

    datom = atom_emb[:, 1, :] - atom_emb[:, 0, :]
    encb = (atom_emb[:, 0, :].sum(0) + vn_emb[0])[None, :]
    combos = jnp.array([[c & 1, (c >> 1) & 1, (c >> 2) & 1]
                        for c in range(8)], f32)
    dbond = bond_emb[:, :, 1, :] - bond_emb[:, :, 0, :]
    cbond = bond_emb[:, :, 0, :].sum(1)
    tables = cbond[:, None, :] + jnp.einsum("cj,ljh->lch", combos, dbond)

    s1 = conv_bng * rs
    w1f = conv_w1 * s1[:, None, :]
    b1f = conv_b1 * s1 + conv_bnb
    s2 = jnp.concatenate([bn_g * rs, jnp.ones((1, H), f32)], 0)
    badd = jnp.concatenate([bn_b, jnp.zeros((1, H), f32)], 0)
    w2f = conv_w2 * s2[:, None, :]
    b2f = conv_b2 * s2 + badd
    sv1 = vn_bn1g[0] * rs
    vw1f = vn_w1[0] * sv1[None, :]
    vb1f = (vn_b1[0] * sv1 + vn_bn1b[0])[None, :]
    sv2 = vn_bn2g[0] * rs
    vw2f = vn_w2[0] * sv2[None, :]
    vb2f = (vn_b2[0] * sv2 + vn_bn2b[0])[None, :]
    eps1 = (1.0 + conv_eps).astype(f32)

    xfp = jnp.zeros((NP, 9), f32).at[:N].set(x.astype(f32))
    batchpad = jnp.concatenate(
        [batch.astype(jnp.int32), jnp.full((NP - N,), -1, jnp.int32)])
    b2d = batchpad.reshape(NP, 1)

    src = edge_index[0].astype(jnp.int32)
    dst = edge_index[1].astype(jnp.int32)
    code = (ex[:, 0] + 2 * ex[:, 1] + 4 * ex[:, 2]).astype(jnp.int32)
    padsrc = (jnp.arange(PAD, dtype=jnp.int32) * 89) % N
    paddst = N + (jnp.arange(PAD, dtype=jnp.int32) % (NP - N))
    padcode = jnp.zeros((PAD,), jnp.int32)

    def tile_pack(a, padvals):
        a2 = a.reshape(TILES, PTE)
        p = jnp.broadcast_to(padvals, (TILES, PAD))
        return jnp.concatenate([a2, p], axis=1).reshape(TILES * WPT, WSZ)

    epk = jnp.stack([tile_pack(src, padsrc), tile_pack(code, padcode),
                     tile_pack(dst, paddst)], axis=1)

    h_a, aug0 = _encbuild(xfp, datom, encb, tables[0])
    p = _sc_aggr(aug0.reshape(NP * 8, H), epk)
    h_c, aug1 = _convpoolbuild(h_a, p, w1f[0], b1f[0][None], w2f[0],
                               b2f[0][None], eps1[0].reshape(1, 1), b2d,
                               vn_emb[0][None], vw1f, vb1f, vw2f, vb2f,
                               tables[1])
    p = _sc_aggr(aug1.reshape(NP * 8, H), epk)
    h_d, aug2 = _convbuild(h_c, p, w1f[1], b1f[1][None], w2f[1],
                           b2f[1][None], eps1[1].reshape(1, 1), tables[2])
    p = _sc_aggr(aug2.reshape(NP * 8, H), epk)
    return _convfinal(h_d, p, w1f[2], b1f[2][None], w2f[2], b2f[2][None],
                      eps1[2].reshape(1, 1), b2d, out_w, out_b[None, :])
